# Initial kernel scaffold; baseline (speedup 1.0000x reference)
import jax
import jax.numpy as jnp
from jax.experimental import pallas as pl


def kernel(ids, token_type_ids, emb_table, type_table, pos_table, ln_scale, ln_bias, W, b):
    raise NotImplementedError("write your pallas kernel here")


# speedup vs baseline: 1.4038x; 1.4038x over previous
import functools

import jax
import jax.numpy as jnp
from jax import lax
from jax.experimental import pallas as pl
from jax.experimental.pallas import tpu as pltpu
from jax.experimental.pallas import tpu_sc as plsc

LN_EPS = 1e-12

_N_TOK = 8192
_D = 128
_H = 2048
_NW = 32
_CPW = 2


def _sc_gather(table, idx2d):
    mesh = plsc.VectorSubcoreMesh(core_axis_name="c", subcore_axis_name="s")

    @functools.partial(
        pl.kernel,
        mesh=mesh,
        out_type=jax.ShapeDtypeStruct((_N_TOK, _D), jnp.float32),
        scratch_types=[
            pltpu.VMEM((_CPW, 128), jnp.int32),
            pltpu.VMEM((_CPW * 128, _D), jnp.float32),
            pltpu.SemaphoreType.DMA,
        ],
    )
    def k(table_hbm, idx_hbm, out_hbm, idx_v, rows_v, sem):
        wid = lax.axis_index("s") * 2 + lax.axis_index("c")
        base = wid * _CPW
        pltpu.sync_copy(idx_hbm.at[pl.ds(base, _CPW)], idx_v)
        copies = []
        for j in range(_CPW):
            copies.append(
                pltpu.async_copy(
                    table_hbm.at[idx_v.at[j]],
                    rows_v.at[pl.ds(j * 128, 128)],
                    sem,
                )
            )
        for cp in copies:
            cp.wait()
        pltpu.sync_copy(rows_v, out_hbm.at[pl.ds(base * 128, _CPW * 128)])

    return k(table, idx2d)


def _tc_tail(g, ttf, type_table, pos_table, ln_scale, ln_bias, W, b):
    TS = 256
    n_blocks = _N_TOK // TS
    pos_blocks = 2048 // TS

    def body(g_ref, tt_ref, type_ref, pos_ref, sc_ref, bi_ref, w_ref,
             bias_ref, o_ref):
        gv = g_ref[...]
        tt = tt_ref[...]
        t0 = type_ref[0:1, :]
        t1 = type_ref[1:2, :]
        te = t0 + tt * (t1 - t0)
        total = gv + te + pos_ref[...]
        mean = jnp.mean(total, axis=-1, keepdims=True)
        cent = total - mean
        var = jnp.mean(cent * cent, axis=-1, keepdims=True)
        xn = cent * lax.rsqrt(var + LN_EPS)
        xn = xn * sc_ref[...] + bi_ref[...]
        o_ref[...] = (
            jnp.dot(xn, w_ref[...], preferred_element_type=jnp.float32)
            + bias_ref[...]
        )

    return pl.pallas_call(
        body,
        grid=(n_blocks,),
        in_specs=[
            pl.BlockSpec((TS, _D), lambda i: (i, 0)),
            pl.BlockSpec((TS, 1), lambda i: (i, 0)),
            pl.BlockSpec((2, _D), lambda i: (0, 0)),
            pl.BlockSpec((TS, _D), lambda i: (i % pos_blocks, 0)),
            pl.BlockSpec((1, _D), lambda i: (0, 0)),
            pl.BlockSpec((1, _D), lambda i: (0, 0)),
            pl.BlockSpec((_D, _H), lambda i: (0, 0)),
            pl.BlockSpec((1, _H), lambda i: (0, 0)),
        ],
        out_specs=pl.BlockSpec((TS, _H), lambda i: (i, 0)),
        out_shape=jax.ShapeDtypeStruct((_N_TOK, _H), jnp.float32),
    )(g, ttf, type_table, pos_table, ln_scale, ln_bias, W, b)


def kernel(ids, token_type_ids, emb_table, type_table, pos_table, ln_scale,
           ln_bias, W, b):
    B, S = ids.shape
    idx2d = ids.astype(jnp.int32).reshape(_N_TOK // 128, 128)
    g = _sc_gather(emb_table, idx2d)
    ttf = token_type_ids.astype(jnp.float32).reshape(_N_TOK, 1)
    hidden = _tc_tail(
        g, ttf, type_table, pos_table,
        ln_scale.reshape(1, _D), ln_bias.reshape(1, _D),
        W, b.reshape(1, _H),
    )
    return hidden.reshape(B, S, _H)


# speedup vs baseline: 1.6278x; 1.1595x over previous
import functools

import jax
import jax.numpy as jnp
from jax import lax
from jax.experimental import pallas as pl
from jax.experimental.pallas import tpu as pltpu
from jax.experimental.pallas import tpu_sc as plsc

LN_EPS = 1e-12

_N_TOK = 8192
_D = 128
_H = 2048
_NW = 32
_CPW = 2


def _sc_gather(table, idx2d):
    mesh = plsc.VectorSubcoreMesh(core_axis_name="c", subcore_axis_name="s")

    @functools.partial(
        pl.kernel,
        mesh=mesh,
        out_type=jax.ShapeDtypeStruct((_N_TOK, _D), jnp.float32),
        scratch_types=[
            pltpu.VMEM((_CPW, 128), jnp.int32),
            pltpu.VMEM((_CPW * 128, _D), jnp.float32),
            pltpu.SemaphoreType.DMA,
        ],
    )
    def k(table_hbm, idx_hbm, out_hbm, idx_v, rows_v, sem):
        wid = lax.axis_index("s") * 2 + lax.axis_index("c")
        base = wid * _CPW
        pltpu.sync_copy(idx_hbm.at[pl.ds(base, _CPW)], idx_v)
        copies = []
        for j in range(_CPW):
            copies.append(
                pltpu.async_copy(
                    table_hbm.at[idx_v.at[j]],
                    rows_v.at[pl.ds(j * 128, 128)],
                    sem,
                )
            )
        for cp in copies:
            cp.wait()
        pltpu.sync_copy(rows_v, out_hbm.at[pl.ds(base * 128, _CPW * 128)])

    return k(table, idx2d)


def _tc_tail(g, ttf, type_table, pos_table, ln_scale, ln_bias, W, b):
    TS = 512
    n_blocks = _N_TOK // TS
    pos_blocks = 2048 // TS

    def body(g_ref, tt_ref, type_ref, pos_ref, sc_ref, bi_ref, w_ref,
             bias_ref, o_ref):
        gv = g_ref[...]
        tt = tt_ref[...]
        t0 = type_ref[0:1, :]
        t1 = type_ref[1:2, :]
        te = t0 + tt * (t1 - t0)
        total = gv + te + pos_ref[...]
        mean = jnp.mean(total, axis=-1, keepdims=True)
        cent = total - mean
        var = jnp.mean(cent * cent, axis=-1, keepdims=True)
        xn = cent * lax.rsqrt(var + LN_EPS)
        xn = xn * sc_ref[...] + bi_ref[...]
        o_ref[...] = (
            jnp.dot(xn, w_ref[...], preferred_element_type=jnp.float32)
            + bias_ref[...]
        )

    return pl.pallas_call(
        body,
        grid=(n_blocks,),
        in_specs=[
            pl.BlockSpec((TS, _D), lambda i: (i, 0)),
            pl.BlockSpec((TS, 1), lambda i: (i, 0)),
            pl.BlockSpec((2, _D), lambda i: (0, 0)),
            pl.BlockSpec((TS, _D), lambda i: (i % pos_blocks, 0)),
            pl.BlockSpec((1, _D), lambda i: (0, 0)),
            pl.BlockSpec((1, _D), lambda i: (0, 0)),
            pl.BlockSpec((_D, _H), lambda i: (0, 0)),
            pl.BlockSpec((1, _H), lambda i: (0, 0)),
        ],
        out_specs=pl.BlockSpec((TS, _H), lambda i: (i, 0)),
        out_shape=jax.ShapeDtypeStruct((_N_TOK, _H), jnp.float32),
    )(g, ttf, type_table, pos_table, ln_scale, ln_bias, W, b)


def kernel(ids, token_type_ids, emb_table, type_table, pos_table, ln_scale,
           ln_bias, W, b):
    B, S = ids.shape
    idx2d = ids.astype(jnp.int32).reshape(_N_TOK // 128, 128)
    g = _sc_gather(emb_table, idx2d)
    ttf = token_type_ids.astype(jnp.float32).reshape(_N_TOK, 1)
    hidden = _tc_tail(
        g, ttf, type_table, pos_table,
        ln_scale.reshape(1, _D), ln_bias.reshape(1, _D),
        W, b.reshape(1, _H),
    )
    return hidden.reshape(B, S, _H)


# speedup vs baseline: 1.7520x; 1.0763x over previous
import functools

import jax
import jax.numpy as jnp
from jax import lax
from jax.experimental import pallas as pl
from jax.experimental.pallas import tpu as pltpu
from jax.experimental.pallas import tpu_sc as plsc

LN_EPS = 1e-12

_N_TOK = 8192
_D = 128
_H = 2048
_NW = 32
_CPW = 2


def _sc_gather(table, idx2d):
    mesh = plsc.VectorSubcoreMesh(core_axis_name="c", subcore_axis_name="s")

    @functools.partial(
        pl.kernel,
        mesh=mesh,
        out_type=jax.ShapeDtypeStruct((_N_TOK, _D), jnp.float32),
        scratch_types=[
            pltpu.VMEM((_CPW, 128), jnp.int32),
            pltpu.VMEM((_CPW * 128, _D), jnp.float32),
            pltpu.SemaphoreType.DMA,
        ],
    )
    def k(table_hbm, idx_hbm, out_hbm, idx_v, rows_v, sem):
        wid = lax.axis_index("s") * 2 + lax.axis_index("c")
        base = wid * _CPW
        pltpu.sync_copy(idx_hbm.at[pl.ds(base, _CPW)], idx_v)
        copies = []
        for j in range(_CPW):
            copies.append(
                pltpu.async_copy(
                    table_hbm.at[idx_v.at[j]],
                    rows_v.at[pl.ds(j * 128, 128)],
                    sem,
                )
            )
        for cp in copies:
            cp.wait()
        pltpu.sync_copy(rows_v, out_hbm.at[pl.ds(base * 128, _CPW * 128)])

    return k(table, idx2d)


def _tc_tail(g, ttf, type_table, pos_table, ln_scale, ln_bias, W, b):
    TS = 1024
    n_blocks = _N_TOK // TS
    pos_blocks = 2048 // TS

    def body(g_ref, tt_ref, type_ref, pos_ref, sc_ref, bi_ref, w_ref,
             bias_ref, o_ref):
        gv = g_ref[...]
        tt = tt_ref[...]
        t0 = type_ref[0:1, :]
        t1 = type_ref[1:2, :]
        te = t0 + tt * (t1 - t0)
        total = gv + te + pos_ref[...]
        mean = jnp.mean(total, axis=-1, keepdims=True)
        cent = total - mean
        var = jnp.mean(cent * cent, axis=-1, keepdims=True)
        xn = cent * lax.rsqrt(var + LN_EPS)
        xn = xn * sc_ref[...] + bi_ref[...]
        o_ref[...] = (
            jnp.dot(xn, w_ref[...], preferred_element_type=jnp.float32)
            + bias_ref[...]
        )

    return pl.pallas_call(
        body,
        grid=(n_blocks,),
        in_specs=[
            pl.BlockSpec((TS, _D), lambda i: (i, 0)),
            pl.BlockSpec((TS, 1), lambda i: (i, 0)),
            pl.BlockSpec((2, _D), lambda i: (0, 0)),
            pl.BlockSpec((TS, _D), lambda i: (i % pos_blocks, 0)),
            pl.BlockSpec((1, _D), lambda i: (0, 0)),
            pl.BlockSpec((1, _D), lambda i: (0, 0)),
            pl.BlockSpec((_D, _H), lambda i: (0, 0)),
            pl.BlockSpec((1, _H), lambda i: (0, 0)),
        ],
        out_specs=pl.BlockSpec((TS, _H), lambda i: (i, 0)),
        out_shape=jax.ShapeDtypeStruct((_N_TOK, _H), jnp.float32),
    )(g, ttf, type_table, pos_table, ln_scale, ln_bias, W, b)


def kernel(ids, token_type_ids, emb_table, type_table, pos_table, ln_scale,
           ln_bias, W, b):
    B, S = ids.shape
    idx2d = ids.astype(jnp.int32).reshape(_N_TOK // 128, 128)
    g = _sc_gather(emb_table, idx2d)
    ttf = token_type_ids.astype(jnp.float32).reshape(_N_TOK, 1)
    hidden = _tc_tail(
        g, ttf, type_table, pos_table,
        ln_scale.reshape(1, _D), ln_bias.reshape(1, _D),
        W, b.reshape(1, _H),
    )
    return hidden.reshape(B, S, _H)
